# baseline (speedup 1.0000x reference)
import functools

import jax
import jax.numpy as jnp
from jax import lax
from jax.experimental import pallas as pl
from jax.experimental.pallas import tpu as pltpu
from jax.experimental.pallas import tpu_sc as plsc

B, S, D = 4, 8192, 768
POOL, LEN, TOPK = 1024, 8, 8
BLK = 64
NS = S // BLK
OUT_ROWS = TOPK * LEN + S


def _tc_body(x_ref, key_ref, out_ref, sim_ref, xsum_ref, knorm_ref):
    b = pl.program_id(0)
    i = pl.program_id(1)

    @pl.when(jnp.logical_and(b == 0, i == 0))
    def _():
        k = key_ref[...]
        ssq = jnp.sum(k * k, axis=1, keepdims=True)
        knorm_ref[...] = k / jnp.maximum(jnp.sqrt(ssq), 1e-12)

    x = x_ref[0]
    out_ref[0] = x
    psum = jnp.sum(x, axis=0, keepdims=True)

    @pl.when(i == 0)
    def _():
        xsum_ref[...] = psum

    @pl.when(i > 0)
    def _():
        xsum_ref[...] = xsum_ref[...] + psum

    @pl.when(i == NS - 1)
    def _():
        xm = xsum_ref[...] * (1.0 / S)
        n = jnp.sqrt(jnp.sum(xm * xm))
        xn = xm / jnp.maximum(n, 1e-12)
        sim_ref[0] = lax.dot_general(
            xn, knorm_ref[...],
            dimension_numbers=(((1,), (1,)), ((), ())),
            preferred_element_type=jnp.float32,
            precision=lax.Precision.HIGHEST,
        )


_tc_call = pl.pallas_call(
    _tc_body,
    grid=(B, NS),
    in_specs=[
        pl.BlockSpec((1, BLK, D), lambda b, i: (b, i, 0)),
        pl.BlockSpec((POOL, D), lambda b, i: (0, 0)),
    ],
    out_specs=[
        pl.BlockSpec((1, BLK, D), lambda b, i: (b, i + 1, 0)),
        pl.BlockSpec((1, 1, POOL), lambda b, i: (b, 0, 0)),
    ],
    out_shape=[
        jax.ShapeDtypeStruct((B, OUT_ROWS, D), jnp.float32),
        jax.ShapeDtypeStruct((B, 1, POOL), jnp.float32),
    ],
    scratch_shapes=[
        pltpu.VMEM((1, D), jnp.float32),
        pltpu.VMEM((POOL, D), jnp.float32),
    ],
)


_NEG = -3.0e38
_mesh = plsc.VectorSubcoreMesh(core_axis_name="c", subcore_axis_name="s")


@functools.partial(
    pl.kernel,
    mesh=_mesh,
    out_type=jax.ShapeDtypeStruct((16,), jnp.float32),
    scratch_types=[
        pltpu.VMEM((POOL,), jnp.float32),
        pltpu.VMEM((16,), jnp.int32),
        pltpu.VMEM((16, LEN, D), jnp.float32),
        pltpu.VMEM((16,), jnp.float32),
        pltpu.VMEM((B * 16,), jnp.float32),
        pltpu.VMEM_SHARED((B * 16,), jnp.float32),
        pltpu.SemaphoreType.DMA,
    ],
)
def _sc_call(sim_hbm, prompt_hbm, big_ref, rsim_out,
             sim_v, idx_v, rows_v, stage_v, parts_v, shared, sem):
    c = lax.axis_index("c")
    s = lax.axis_index("s")
    active = jnp.logical_and(c == 0, s < B)
    lanes = lax.iota(jnp.int32, 16)

    def _splat_reduce(v, op):
        for k in (1, 2, 4, 8):
            v = op(v, v.at[lanes ^ k].get(mode="promise_in_bounds"))
        return v

    @pl.when(active)
    def _():
        bb = s
        pltpu.sync_copy(sim_hbm.at[bb], sim_v)
        idxs = jnp.zeros((16,), jnp.int32)
        rsum = jnp.zeros((16,), jnp.float32)
        pg = jnp.full((16,), jnp.inf, jnp.float32)
        pij = jnp.full((16,), -1, jnp.int32)
        for j in range(TOPK):
            def scan(i, carry, pg=pg, pij=pij):
                m, am = carry
                v = sim_v[pl.ds(i * 16, 16)]
                gi = lanes + i * 16
                elig = jnp.logical_or(v < pg,
                                      jnp.logical_and(v == pg, gi > pij))
                upd = jnp.logical_and(elig, v > m)
                return jnp.where(upd, v, m), jnp.where(upd, gi, am)
            m, am = lax.fori_loop(
                0, POOL // 16, scan,
                (jnp.full((16,), _NEG, jnp.float32), jnp.zeros((16,), jnp.int32)),
            )
            gmax = _splat_reduce(m, jnp.maximum)
            cand = jnp.where(m == gmax, am, POOL)
            ij = _splat_reduce(cand, jnp.minimum)
            idxs = jnp.where(lanes == j, ij, idxs)
            rsum = rsum + gmax
            pg, pij = gmax, ij
        idx_v[...] = idxs
        pltpu.async_copy(prompt_hbm.at[idx_v], rows_v, sem).wait()
        pltpu.sync_copy(rows_v.at[pl.ds(0, TOPK)], big_ref.at[bb, pl.ds(0, TOPK)])
        stage_v[...] = rsum
        pltpu.sync_copy(stage_v, shared.at[pl.ds(bb * 16, 16)])

    plsc.subcore_barrier()

    @pl.when(jnp.logical_and(c == 0, s == 0))
    def _():
        pltpu.sync_copy(shared, parts_v)
        tot = (parts_v[pl.ds(0, 16)] + parts_v[pl.ds(16, 16)]
               + parts_v[pl.ds(32, 16)] + parts_v[pl.ds(48, 16)]) * (1.0 / B)
        stage_v[...] = tot
        pltpu.sync_copy(stage_v, rsim_out)


def kernel(x_embed, prompt, prompt_key):
    big, sim = _tc_call(x_embed, prompt_key)
    sim = sim.reshape(B, POOL)
    big4 = big.reshape(B, OUT_ROWS // LEN, LEN, D)
    prompt3 = prompt.reshape(POOL, LEN, D)
    ref = jax.new_ref(big4)
    rsim_vec = _sc_call(sim, prompt3, ref)
    out = ref[...].reshape(B, OUT_ROWS, D)
    return out, rsim_vec[0]


# speedup vs baseline: 3.5649x; 3.5649x over previous
import functools

import jax
import jax.numpy as jnp
from jax import lax
from jax.experimental import pallas as pl
from jax.experimental.pallas import tpu as pltpu
from jax.experimental.pallas import tpu_sc as plsc

B, S, D = 4, 8192, 768
POOL, LEN, TOPK = 1024, 8, 8
BLK = 2048
NS = S // BLK
HEAD = TOPK * LEN
OUT_ROWS = HEAD + S


def _tc_body(x_ref, key_ref, out_ref, sim_ref, xsum_ref, knorm_ref, sem):
    b = pl.program_id(0)
    i = pl.program_id(1)

    cp = pltpu.make_async_copy(
        x_ref.at[0], out_ref.at[b, pl.ds(HEAD + i * BLK, BLK)], sem)
    cp.start()

    @pl.when(jnp.logical_and(b == 0, i == 0))
    def _():
        k = key_ref[...]
        ssq = jnp.sum(k * k, axis=1, keepdims=True)
        knorm_ref[...] = k / jnp.maximum(jnp.sqrt(ssq), 1e-12)

    psum = jnp.sum(x_ref[0], axis=0, keepdims=True)

    @pl.when(i == 0)
    def _():
        xsum_ref[...] = psum

    @pl.when(i > 0)
    def _():
        xsum_ref[...] = xsum_ref[...] + psum

    @pl.when(i == NS - 1)
    def _():
        xm = xsum_ref[...] * (1.0 / S)
        n = jnp.sqrt(jnp.sum(xm * xm))
        xn = xm / jnp.maximum(n, 1e-12)
        sim_ref[0] = lax.dot_general(
            xn, knorm_ref[...],
            dimension_numbers=(((1,), (1,)), ((), ())),
            preferred_element_type=jnp.float32,
            precision=lax.Precision.HIGHEST,
        )

    cp.wait()


_tc_call = pl.pallas_call(
    _tc_body,
    grid=(B, NS),
    in_specs=[
        pl.BlockSpec((1, BLK, D), lambda b, i: (b, i, 0)),
        pl.BlockSpec((POOL, D), lambda b, i: (0, 0)),
    ],
    out_specs=[
        pl.BlockSpec(memory_space=pl.ANY),
        pl.BlockSpec((1, 1, POOL), lambda b, i: (b, 0, 0)),
    ],
    out_shape=[
        jax.ShapeDtypeStruct((B, OUT_ROWS, D), jnp.float32),
        jax.ShapeDtypeStruct((B, 1, POOL), jnp.float32),
    ],
    scratch_shapes=[
        pltpu.VMEM((1, D), jnp.float32),
        pltpu.VMEM((POOL, D), jnp.float32),
        pltpu.SemaphoreType.DMA,
    ],
)


_NEG = -3.0e38
_mesh = plsc.VectorSubcoreMesh(core_axis_name="c", subcore_axis_name="s")


@functools.partial(
    pl.kernel,
    mesh=_mesh,
    out_type=jax.ShapeDtypeStruct((16,), jnp.float32),
    scratch_types=[
        pltpu.VMEM((POOL,), jnp.float32),
        pltpu.VMEM((16,), jnp.int32),
        pltpu.VMEM((16, LEN, D), jnp.float32),
        pltpu.VMEM((16,), jnp.float32),
        pltpu.VMEM((B * 16,), jnp.float32),
        pltpu.VMEM_SHARED((B * 16,), jnp.float32),
        pltpu.SemaphoreType.DMA,
    ],
)
def _sc_call(sim_hbm, prompt_hbm, big_ref, rsim_out,
             sim_v, idx_v, rows_v, stage_v, parts_v, shared, sem):
    c = lax.axis_index("c")
    s = lax.axis_index("s")
    active = jnp.logical_and(c == 0, s < B)
    lanes = lax.iota(jnp.int32, 16)

    def _splat_reduce(v, op):
        for k in (1, 2, 4, 8):
            v = op(v, v.at[lanes ^ k].get(mode="promise_in_bounds"))
        return v

    @pl.when(active)
    def _():
        bb = s
        pltpu.sync_copy(sim_hbm.at[bb], sim_v)
        idxs = jnp.zeros((16,), jnp.int32)
        rsum = jnp.zeros((16,), jnp.float32)
        pg = jnp.full((16,), jnp.inf, jnp.float32)
        pij = jnp.full((16,), -1, jnp.int32)
        for j in range(TOPK):
            def scan(i, carry, pg=pg, pij=pij):
                m, am = carry
                v = sim_v[pl.ds(i * 16, 16)]
                gi = lanes + i * 16
                elig = jnp.logical_or(v < pg,
                                      jnp.logical_and(v == pg, gi > pij))
                upd = jnp.logical_and(elig, v > m)
                return jnp.where(upd, v, m), jnp.where(upd, gi, am)
            m, am = lax.fori_loop(
                0, POOL // 16, scan,
                (jnp.full((16,), _NEG, jnp.float32), jnp.zeros((16,), jnp.int32)),
            )
            gmax = _splat_reduce(m, jnp.maximum)
            cand = jnp.where(m == gmax, am, POOL)
            ij = _splat_reduce(cand, jnp.minimum)
            idxs = jnp.where(lanes == j, ij, idxs)
            rsum = rsum + gmax
            pg, pij = gmax, ij
        idx_v[...] = idxs
        pltpu.async_copy(prompt_hbm.at[idx_v], rows_v, sem).wait()
        pltpu.sync_copy(rows_v.at[pl.ds(0, TOPK)], big_ref.at[bb, pl.ds(0, TOPK)])
        stage_v[...] = rsum
        pltpu.sync_copy(stage_v, shared.at[pl.ds(bb * 16, 16)])

    plsc.subcore_barrier()

    @pl.when(jnp.logical_and(c == 0, s == 0))
    def _():
        pltpu.sync_copy(shared, parts_v)
        tot = (parts_v[pl.ds(0, 16)] + parts_v[pl.ds(16, 16)]
               + parts_v[pl.ds(32, 16)] + parts_v[pl.ds(48, 16)]) * (1.0 / B)
        stage_v[...] = tot
        pltpu.sync_copy(stage_v, rsim_out)


def kernel(x_embed, prompt, prompt_key):
    big, sim = _tc_call(x_embed, prompt_key)
    sim = sim.reshape(B, POOL)
    big4 = big.reshape(B, OUT_ROWS // LEN, LEN, D)
    prompt3 = prompt.reshape(POOL, LEN, D)
    ref = jax.new_ref(big4)
    rsim_vec = _sc_call(sim, prompt3, ref)
    out = ref[...].reshape(B, OUT_ROWS, D)
    return out, rsim_vec[0]


# speedup vs baseline: 3.6074x; 1.0119x over previous
import functools

import jax
import jax.numpy as jnp
from jax import lax
from jax.experimental import pallas as pl
from jax.experimental.pallas import tpu as pltpu
from jax.experimental.pallas import tpu_sc as plsc

B, S, D = 4, 8192, 768
POOL, LEN, TOPK = 1024, 8, 8
BLK = 2048
NS = S // BLK
HEAD = TOPK * LEN
OUT_ROWS = HEAD + S


def _tc_body(x_ref, key_ref, out_ref, sim_ref, xsum_ref, knorm_ref, sem):
    b = pl.program_id(0)
    i = pl.program_id(1)

    cp = pltpu.make_async_copy(
        x_ref.at[0], out_ref.at[b, pl.ds(HEAD + i * BLK, BLK)], sem)
    cp.start()

    @pl.when(jnp.logical_and(b == 0, i == 0))
    def _():
        k = key_ref[...]
        ssq = jnp.sum(k * k, axis=1, keepdims=True)
        knorm_ref[...] = k / jnp.maximum(jnp.sqrt(ssq), 1e-12)

    psum = jnp.sum(x_ref[0], axis=0, keepdims=True)

    @pl.when(i == 0)
    def _():
        xsum_ref[...] = psum

    @pl.when(i > 0)
    def _():
        xsum_ref[...] = xsum_ref[...] + psum

    @pl.when(i == NS - 1)
    def _():
        xm = xsum_ref[...] * (1.0 / S)
        n = jnp.sqrt(jnp.sum(xm * xm))
        xn = xm / jnp.maximum(n, 1e-12)
        sim_ref[0] = lax.dot_general(
            xn, knorm_ref[...],
            dimension_numbers=(((1,), (1,)), ((), ())),
            preferred_element_type=jnp.float32,
            precision=lax.Precision.HIGHEST,
        )

    cp.wait()


_tc_call = pl.pallas_call(
    _tc_body,
    grid=(B, NS),
    in_specs=[
        pl.BlockSpec((1, BLK, D), lambda b, i: (b, i, 0)),
        pl.BlockSpec((POOL, D), lambda b, i: (0, 0)),
    ],
    out_specs=[
        pl.BlockSpec(memory_space=pl.ANY),
        pl.BlockSpec((1, 1, POOL), lambda b, i: (b, 0, 0)),
    ],
    out_shape=[
        jax.ShapeDtypeStruct((B, OUT_ROWS, D), jnp.float32),
        jax.ShapeDtypeStruct((B, 1, POOL), jnp.float32),
    ],
    scratch_shapes=[
        pltpu.VMEM((1, D), jnp.float32),
        pltpu.VMEM((POOL, D), jnp.float32),
        pltpu.SemaphoreType.DMA,
    ],
)


_NEG = -3.0e38
_mesh = plsc.VectorSubcoreMesh(core_axis_name="c", subcore_axis_name="s")


_NT = 4
_SLICE = POOL // _NT


@functools.partial(
    pl.kernel,
    mesh=_mesh,
    out_type=jax.ShapeDtypeStruct((16,), jnp.float32),
    scratch_types=[
        pltpu.VMEM((_SLICE,), jnp.float32),
        pltpu.VMEM((16,), jnp.int32),
        pltpu.VMEM((16, LEN, D), jnp.float32),
        pltpu.VMEM((16,), jnp.float32),
        pltpu.VMEM((_NT * 16,), jnp.float32),
        pltpu.VMEM((_NT * 16,), jnp.int32),
        pltpu.VMEM((B * 16,), jnp.float32),
        pltpu.VMEM_SHARED((B * _NT * 16,), jnp.float32),
        pltpu.VMEM_SHARED((B * _NT * 16,), jnp.int32),
        pltpu.VMEM_SHARED((B * 16,), jnp.float32),
        pltpu.SemaphoreType.DMA,
    ],
)
def _sc_call(sim_hbm, prompt_hbm, big_ref, rsim_out,
             sim_v, idx_v, rows_v, stage_v, mv_v, mi_v, parts_v,
             sh_v, sh_i, sh_r, sem):
    c = lax.axis_index("c")
    s = lax.axis_index("s")
    lanes = lax.iota(jnp.int32, 16)

    def _splat_reduce(v, op):
        for k in (1, 2, 4, 8):
            v = op(v, v.at[lanes ^ k].get(mode="promise_in_bounds"))
        return v

    @pl.when(c == 0)
    def _():
        bb = s // _NT
        part = s % _NT
        base = part * _SLICE
        pltpu.sync_copy(sim_hbm.at[bb, pl.ds(base, _SLICE)], sim_v)
        vals = jnp.full((16,), _NEG, jnp.float32)
        idxs = jnp.full((16,), POOL, jnp.int32)
        pg = jnp.full((16,), jnp.inf, jnp.float32)
        pij = jnp.full((16,), -1, jnp.int32)
        for j in range(TOPK):
            def scan(i, carry, pg=pg, pij=pij, base=base):
                m, am = carry
                v = sim_v[pl.ds(i * 16, 16)]
                gi = lanes + (base + i * 16)
                elig = jnp.logical_or(v < pg,
                                      jnp.logical_and(v == pg, gi > pij))
                upd = jnp.logical_and(elig, v > m)
                return jnp.where(upd, v, m), jnp.where(upd, gi, am)
            m, am = lax.fori_loop(
                0, _SLICE // 16, scan,
                (jnp.full((16,), _NEG, jnp.float32), jnp.zeros((16,), jnp.int32)),
            )
            gmax = _splat_reduce(m, jnp.maximum)
            cand = jnp.where(m == gmax, am, POOL)
            ij = _splat_reduce(cand, jnp.minimum)
            vals = jnp.where(lanes == j, gmax, vals)
            idxs = jnp.where(lanes == j, ij, idxs)
            pg, pij = gmax, ij
        stage_v[...] = vals
        pltpu.sync_copy(stage_v, sh_v.at[pl.ds(s * 16, 16)])
        idx_v[...] = idxs
        pltpu.sync_copy(idx_v, sh_i.at[pl.ds(s * 16, 16)])

    plsc.subcore_barrier()

    @pl.when(jnp.logical_and(c == 0, s % _NT == 0))
    def _():
        bb = s // _NT
        pltpu.sync_copy(sh_v.at[pl.ds(bb * _NT * 16, _NT * 16)], mv_v)
        pltpu.sync_copy(sh_i.at[pl.ds(bb * _NT * 16, _NT * 16)], mi_v)
        idxs = jnp.zeros((16,), jnp.int32)
        rsum = jnp.zeros((16,), jnp.float32)
        pg = jnp.full((16,), jnp.inf, jnp.float32)
        pij = jnp.full((16,), -1, jnp.int32)
        for j in range(TOPK):
            m = jnp.full((16,), _NEG, jnp.float32)
            am = jnp.full((16,), POOL, jnp.int32)
            for t in range(_NT):
                v = mv_v[pl.ds(t * 16, 16)]
                gi = mi_v[pl.ds(t * 16, 16)]
                elig = jnp.logical_or(v < pg,
                                      jnp.logical_and(v == pg, gi > pij))
                upd = jnp.logical_and(
                    elig,
                    jnp.logical_or(v > m,
                                   jnp.logical_and(v == m, gi < am)))
                m = jnp.where(upd, v, m)
                am = jnp.where(upd, gi, am)
            gmax = _splat_reduce(m, jnp.maximum)
            cand = jnp.where(m == gmax, am, POOL)
            ij = _splat_reduce(cand, jnp.minimum)
            idxs = jnp.where(lanes == j, ij, idxs)
            rsum = rsum + gmax
            pg, pij = gmax, ij
        idx_v[...] = idxs
        pltpu.async_copy(prompt_hbm.at[idx_v], rows_v, sem).wait()
        pltpu.sync_copy(rows_v.at[pl.ds(0, TOPK)], big_ref.at[bb, pl.ds(0, TOPK)])
        stage_v[...] = rsum
        pltpu.sync_copy(stage_v, sh_r.at[pl.ds(bb * 16, 16)])

    plsc.subcore_barrier()

    @pl.when(jnp.logical_and(c == 0, s == 0))
    def _():
        pltpu.sync_copy(sh_r, parts_v)
        tot = (parts_v[pl.ds(0, 16)] + parts_v[pl.ds(16, 16)]
               + parts_v[pl.ds(32, 16)] + parts_v[pl.ds(48, 16)]) * (1.0 / B)
        stage_v[...] = tot
        pltpu.sync_copy(stage_v, rsim_out)


def kernel(x_embed, prompt, prompt_key):
    big, sim = _tc_call(x_embed, prompt_key)
    sim = sim.reshape(B, POOL)
    big4 = big.reshape(B, OUT_ROWS // LEN, LEN, D)
    prompt3 = prompt.reshape(POOL, LEN, D)
    ref = jax.new_ref(big4)
    rsim_vec = _sc_call(sim, prompt3, ref)
    out = ref[...].reshape(B, OUT_ROWS, D)
    return out, rsim_vec[0]


# speedup vs baseline: 3.9155x; 1.0854x over previous
import functools

import jax
import jax.numpy as jnp
from jax import lax
from jax.experimental import pallas as pl
from jax.experimental.pallas import tpu as pltpu
from jax.experimental.pallas import tpu_sc as plsc

B, S, D = 4, 8192, 768
POOL, LEN, TOPK = 1024, 8, 8
BLK = 4096
NS = S // BLK
HEAD = TOPK * LEN
OUT_ROWS = HEAD + S


def _tc_body(x_ref, key_ref, out_ref, sim_ref, xsum_ref, knorm_ref, sem):
    b = pl.program_id(0)
    i = pl.program_id(1)

    cp = pltpu.make_async_copy(
        x_ref.at[0], out_ref.at[b, pl.ds(HEAD + i * BLK, BLK)], sem)
    cp.start()

    @pl.when(jnp.logical_and(b == 0, i == 0))
    def _():
        k = key_ref[...]
        ssq = jnp.sum(k * k, axis=1, keepdims=True)
        knorm_ref[...] = k / jnp.maximum(jnp.sqrt(ssq), 1e-12)

    psum = jnp.sum(x_ref[0], axis=0, keepdims=True)

    @pl.when(i == 0)
    def _():
        xsum_ref[...] = psum

    @pl.when(i > 0)
    def _():
        xsum_ref[...] = xsum_ref[...] + psum

    @pl.when(i == NS - 1)
    def _():
        xm = xsum_ref[...] * (1.0 / S)
        n = jnp.sqrt(jnp.sum(xm * xm))
        xn = xm / jnp.maximum(n, 1e-12)
        sim_ref[0] = lax.dot_general(
            xn, knorm_ref[...],
            dimension_numbers=(((1,), (1,)), ((), ())),
            preferred_element_type=jnp.float32,
            precision=lax.Precision.HIGHEST,
        )

    cp.wait()


_tc_call = pl.pallas_call(
    _tc_body,
    grid=(B, NS),
    in_specs=[
        pl.BlockSpec((1, BLK, D), lambda b, i: (b, i, 0)),
        pl.BlockSpec((POOL, D), lambda b, i: (0, 0)),
    ],
    out_specs=[
        pl.BlockSpec(memory_space=pl.ANY),
        pl.BlockSpec((1, 1, POOL), lambda b, i: (b, 0, 0)),
    ],
    out_shape=[
        jax.ShapeDtypeStruct((B, OUT_ROWS, D), jnp.float32),
        jax.ShapeDtypeStruct((B, 1, POOL), jnp.float32),
    ],
    scratch_shapes=[
        pltpu.VMEM((1, D), jnp.float32),
        pltpu.VMEM((POOL, D), jnp.float32),
        pltpu.SemaphoreType.DMA,
    ],
)


_NEG = -3.0e38
_mesh = plsc.VectorSubcoreMesh(core_axis_name="c", subcore_axis_name="s")


_NT = 4
_SLICE = POOL // _NT


@functools.partial(
    pl.kernel,
    mesh=_mesh,
    out_type=jax.ShapeDtypeStruct((16,), jnp.float32),
    scratch_types=[
        pltpu.VMEM((_SLICE,), jnp.float32),
        pltpu.VMEM((16,), jnp.int32),
        pltpu.VMEM((TOPK, LEN, D), jnp.float32),
        pltpu.VMEM((16,), jnp.float32),
        pltpu.VMEM((_NT * 16,), jnp.float32),
        pltpu.VMEM((_NT * 16,), jnp.int32),
        pltpu.VMEM((B * 16,), jnp.float32),
        pltpu.VMEM_SHARED((B * _NT * 16,), jnp.float32),
        pltpu.VMEM_SHARED((B * _NT * 16,), jnp.int32),
        pltpu.VMEM_SHARED((B * 16,), jnp.float32),
        pltpu.SemaphoreType.DMA,
    ],
)
def _sc_call(sim_hbm, prompt_hbm, big_ref, rsim_out,
             sim_v, idx_v, rows_v, stage_v, mv_v, mi_v, parts_v,
             sh_v, sh_i, sh_r, sem):
    c = lax.axis_index("c")
    s = lax.axis_index("s")
    lanes = lax.iota(jnp.int32, 16)

    def _splat_reduce(v, op):
        for k in (1, 2, 4, 8):
            v = op(v, v.at[lanes ^ k].get(mode="promise_in_bounds"))
        return v

    @pl.when(c == 0)
    def _():
        bb = s // _NT
        part = s % _NT
        base = part * _SLICE
        pltpu.sync_copy(sim_hbm.at[bb, pl.ds(base, _SLICE)], sim_v)
        vals = jnp.full((16,), _NEG, jnp.float32)
        idxs = jnp.full((16,), POOL, jnp.int32)
        pg = jnp.full((16,), jnp.inf, jnp.float32)
        pij = jnp.full((16,), -1, jnp.int32)
        for j in range(TOPK):
            def scan(i, carry, pg=pg, pij=pij, base=base):
                m, am = carry
                v = sim_v[pl.ds(i * 16, 16)]
                gi = lanes + (base + i * 16)
                elig = jnp.logical_or(v < pg,
                                      jnp.logical_and(v == pg, gi > pij))
                upd = jnp.logical_and(elig, v > m)
                return jnp.where(upd, v, m), jnp.where(upd, gi, am)
            m, am = lax.fori_loop(
                0, _SLICE // 16, scan,
                (jnp.full((16,), _NEG, jnp.float32), jnp.zeros((16,), jnp.int32)),
            )
            gmax = _splat_reduce(m, jnp.maximum)
            cand = jnp.where(m == gmax, am, POOL)
            ij = _splat_reduce(cand, jnp.minimum)
            vals = jnp.where(lanes == j, gmax, vals)
            idxs = jnp.where(lanes == j, ij, idxs)
            pg, pij = gmax, ij
        stage_v[...] = vals
        pltpu.sync_copy(stage_v, sh_v.at[pl.ds(s * 16, 16)])
        idx_v[...] = idxs
        pltpu.sync_copy(idx_v, sh_i.at[pl.ds(s * 16, 16)])

    plsc.subcore_barrier()

    @pl.when(jnp.logical_and(c == 0, s % _NT == 0))
    def _():
        bb = s // _NT
        c1 = pltpu.async_copy(sh_v.at[pl.ds(bb * _NT * 16, _NT * 16)], mv_v, sem)
        c2 = pltpu.async_copy(sh_i.at[pl.ds(bb * _NT * 16, _NT * 16)], mi_v, sem)
        c1.wait()
        c2.wait()
        idxs = jnp.zeros((16,), jnp.int32)
        rsum = jnp.zeros((16,), jnp.float32)
        pg = jnp.full((16,), jnp.inf, jnp.float32)
        pij = jnp.full((16,), -1, jnp.int32)
        for j in range(TOPK):
            m = jnp.full((16,), _NEG, jnp.float32)
            am = jnp.full((16,), POOL, jnp.int32)
            for t in range(_NT):
                v = mv_v[pl.ds(t * 16, 16)]
                gi = mi_v[pl.ds(t * 16, 16)]
                elig = jnp.logical_or(v < pg,
                                      jnp.logical_and(v == pg, gi > pij))
                upd = jnp.logical_and(
                    elig,
                    jnp.logical_or(v > m,
                                   jnp.logical_and(v == m, gi < am)))
                m = jnp.where(upd, v, m)
                am = jnp.where(upd, gi, am)
            gmax = _splat_reduce(m, jnp.maximum)
            cand = jnp.where(m == gmax, am, POOL)
            ij = _splat_reduce(cand, jnp.minimum)
            idxs = jnp.where(lanes == j, ij, idxs)
            rsum = rsum + gmax
            pg, pij = gmax, ij
        idx_v[...] = idxs
        pltpu.async_copy(prompt_hbm.at[idx_v.at[pl.ds(0, TOPK)]], rows_v,
                         sem).wait()
        pltpu.sync_copy(rows_v, big_ref.at[bb, pl.ds(0, TOPK)])
        stage_v[...] = rsum
        pltpu.sync_copy(stage_v, sh_r.at[pl.ds(bb * 16, 16)])

    plsc.subcore_barrier()

    @pl.when(jnp.logical_and(c == 0, s == 0))
    def _():
        pltpu.sync_copy(sh_r, parts_v)
        tot = (parts_v[pl.ds(0, 16)] + parts_v[pl.ds(16, 16)]
               + parts_v[pl.ds(32, 16)] + parts_v[pl.ds(48, 16)]) * (1.0 / B)
        stage_v[...] = tot
        pltpu.sync_copy(stage_v, rsim_out)


def kernel(x_embed, prompt, prompt_key):
    big, sim = _tc_call(x_embed, prompt_key)
    sim = sim.reshape(B, POOL)
    big4 = big.reshape(B, OUT_ROWS // LEN, LEN, D)
    prompt3 = prompt.reshape(POOL, LEN, D)
    ref = jax.new_ref(big4)
    rsim_vec = _sc_call(sim, prompt3, ref)
    out = ref[...].reshape(B, OUT_ROWS, D)
    return out, rsim_vec[0]
